# baseline (speedup 1.0000x reference)
import functools

import jax
import jax.numpy as jnp
from jax import lax
from jax.experimental import pallas as pl
from jax.experimental.pallas import tpu as pltpu
from jax.experimental.pallas import tpu_sc as plsc

D = 2048
E = 8
N = 4096
TOP_K = 2

_BN = 512
_BW = 1024
_JW = D // _BW
_L = 16


def _sc_gate_kernel(x_hbm, gwf_hbm, gb_hbm, idx_hbm, w_hbm,
                    xv, gwv, gbv, oi_v, ow_v):
    c = lax.axis_index("c")
    s = lax.axis_index("s")

    @pl.when((c == 0) & (s == 0))
    def _():
        pltpu.sync_copy(x_hbm.at[0], xv)
        pltpu.sync_copy(gwf_hbm, gwv)
        pltpu.sync_copy(gb_hbm, gbv)
        lanes = lax.iota(jnp.int32, _L)

        def body(k, acc):
            xc = xv[pl.ds(k * _L, _L)]
            for r in range(_L // 2):
                sc = jnp.where(lanes < E, xc[2 * r], xc[2 * r + 1])
                acc = acc + sc * gwv[pl.ds((k * (_L // 2) + r) * _L, _L)]
            return acc

        acc = lax.fori_loop(0, D // _L, body, jnp.zeros((_L,), jnp.float32))
        logv = jnp.where(lanes < E, gbv[...], -jnp.inf)
        for e in range(E):
            logv = jnp.where(lanes == e, logv + acc[e] + acc[e + E], logv)

        gdn = lax.GatherDimensionNumbers(offset_dims=(),
                                         collapsed_slice_dims=(0,),
                                         start_index_map=(0,))

        def rot(v, sh):
            idx = ((lanes + sh) & (_L - 1))[:, None]
            return lax.gather(v, idx, dimension_numbers=gdn,
                              slice_sizes=(1,),
                              mode=lax.GatherScatterMode.PROMISE_IN_BOUNDS)

        mv = logv
        for sh in (8, 4, 2, 1):
            mv = jnp.maximum(mv, rot(mv, sh))
        ev = jnp.exp(logv - mv)
        sv = ev
        for sh in (8, 4, 2, 1):
            sv = sv + rot(sv, sh)
        p = ev / sv

        def vmax(v):
            for sh in (8, 4, 2, 1):
                v = jnp.maximum(v, rot(v, sh))
            return v

        def vmin(v):
            for sh in (8, 4, 2, 1):
                v = jnp.minimum(v, rot(v, sh))
            return v

        m1 = vmax(p)
        a1 = vmin(jnp.where(p == m1, lanes, _L))
        p2 = jnp.where(lanes == a1, -1.0, p)
        m2 = vmax(p2)
        a2 = vmin(jnp.where(p2 == m2, lanes, _L))
        oi_v[...] = jnp.where(lanes == 0, a1, a2)
        ow_v[...] = jnp.where(lanes == 0, m1, m2)
        pltpu.sync_copy(oi_v, idx_hbm)
        pltpu.sync_copy(ow_v, w_hbm)


def _mm_kernel(idx_ref, w_ref, x_ref, w0_ref, w1_ref, b0_ref, b1_ref, o_ref,
               wc_ref, bc_ref):
    del idx_ref
    t = pl.program_id(0)
    w0 = w_ref[0]
    w1 = w_ref[1]

    @pl.when(t < _JW)
    def _():
        wc_ref[:, pl.ds(t * _BW, _BW)] = (
            w0 * w0_ref[0] + w1 * w1_ref[0]).astype(jnp.bfloat16)

    @pl.when(t == 0)
    def _():
        bc_ref[...] = w0 * b0_ref[0] + w1 * b1_ref[0]

    @pl.when(t >= _JW)
    def _():
        acc = jnp.dot(x_ref[...].astype(jnp.bfloat16), wc_ref[...],
                      preferred_element_type=jnp.float32)
        o_ref[...] = acc + bc_ref[...]


@functools.partial(jax.jit, static_argnames=())
def kernel(x, gate_W, gate_b, experts_W, experts_b):
    sc_gate = pl.kernel(
        _sc_gate_kernel,
        out_type=[
            jax.ShapeDtypeStruct((_L,), jnp.int32),
            jax.ShapeDtypeStruct((_L,), jnp.float32),
        ],
        mesh=plsc.VectorSubcoreMesh(core_axis_name="c", subcore_axis_name="s",
                                    num_cores=1, num_subcores=1),
        scratch_types=[
            pltpu.VMEM((D,), jnp.float32),
            pltpu.VMEM((D * E,), jnp.float32),
            pltpu.VMEM((_L,), jnp.float32),
            pltpu.VMEM((_L,), jnp.int32),
            pltpu.VMEM((_L,), jnp.float32),
        ],
    )
    idx16, w16 = sc_gate(x, gate_W.reshape(-1),
                         jnp.pad(gate_b, (0, _L - E)))

    grid = (_JW + N // _BN,)
    out = pl.pallas_call(
        _mm_kernel,
        grid_spec=pltpu.PrefetchScalarGridSpec(
            num_scalar_prefetch=2,
            grid=grid,
            in_specs=[
                pl.BlockSpec((_BN, D),
                             lambda t, idx, w: (jnp.maximum(t - _JW, 0), 0)),
                pl.BlockSpec((1, D, _BW),
                             lambda t, idx, w: (idx[0], 0,
                                                jnp.minimum(t, _JW - 1))),
                pl.BlockSpec((1, D, _BW),
                             lambda t, idx, w: (idx[1], 0,
                                                jnp.minimum(t, _JW - 1))),
                pl.BlockSpec((1, 1, D), lambda t, idx, w: (idx[0], 0, 0)),
                pl.BlockSpec((1, 1, D), lambda t, idx, w: (idx[1], 0, 0)),
            ],
            out_specs=pl.BlockSpec((_BN, D),
                                   lambda t, idx, w: (jnp.maximum(t - _JW, 0),
                                                      0)),
            scratch_shapes=[pltpu.VMEM((D, D), jnp.bfloat16),
                            pltpu.VMEM((1, D), jnp.float32)],
        ),
        out_shape=jax.ShapeDtypeStruct((N, D), jnp.float32),
        compiler_params=pltpu.CompilerParams(
            dimension_semantics=("arbitrary",),
        ),
    )(idx16, w16, x, experts_W, experts_W,
      experts_b.reshape(E, 1, D), experts_b.reshape(E, 1, D))
    return out


# speedup vs baseline: 1.0059x; 1.0059x over previous
import functools

import jax
import jax.numpy as jnp
from jax import lax
from jax.experimental import pallas as pl
from jax.experimental.pallas import tpu as pltpu
from jax.experimental.pallas import tpu_sc as plsc

D = 2048
E = 8
N = 4096
TOP_K = 2

_BN = 512
_BW = 512
_JW = D // _BW
_L = 16


def _sc_gate_kernel(x_hbm, gwf_hbm, gb_hbm, idx_hbm, w_hbm,
                    xv, gwv, gbv, oi_v, ow_v):
    c = lax.axis_index("c")
    s = lax.axis_index("s")

    @pl.when((c == 0) & (s == 0))
    def _():
        pltpu.sync_copy(x_hbm.at[0], xv)
        pltpu.sync_copy(gwf_hbm, gwv)
        pltpu.sync_copy(gb_hbm, gbv)
        lanes = lax.iota(jnp.int32, _L)

        def body(k, acc):
            xc = xv[pl.ds(k * _L, _L)]
            for r in range(_L // 2):
                sc = jnp.where(lanes < E, xc[2 * r], xc[2 * r + 1])
                acc = acc + sc * gwv[pl.ds((k * (_L // 2) + r) * _L, _L)]
            return acc

        acc = lax.fori_loop(0, D // _L, body, jnp.zeros((_L,), jnp.float32))
        logv = jnp.where(lanes < E, gbv[...], -jnp.inf)
        for e in range(E):
            logv = jnp.where(lanes == e, logv + acc[e] + acc[e + E], logv)

        gdn = lax.GatherDimensionNumbers(offset_dims=(),
                                         collapsed_slice_dims=(0,),
                                         start_index_map=(0,))

        def rot(v, sh):
            idx = ((lanes + sh) & (_L - 1))[:, None]
            return lax.gather(v, idx, dimension_numbers=gdn,
                              slice_sizes=(1,),
                              mode=lax.GatherScatterMode.PROMISE_IN_BOUNDS)

        mv = logv
        for sh in (8, 4, 2, 1):
            mv = jnp.maximum(mv, rot(mv, sh))
        ev = jnp.exp(logv - mv)
        sv = ev
        for sh in (8, 4, 2, 1):
            sv = sv + rot(sv, sh)
        p = ev / sv

        def vmax(v):
            for sh in (8, 4, 2, 1):
                v = jnp.maximum(v, rot(v, sh))
            return v

        def vmin(v):
            for sh in (8, 4, 2, 1):
                v = jnp.minimum(v, rot(v, sh))
            return v

        m1 = vmax(p)
        a1 = vmin(jnp.where(p == m1, lanes, _L))
        p2 = jnp.where(lanes == a1, -1.0, p)
        m2 = vmax(p2)
        a2 = vmin(jnp.where(p2 == m2, lanes, _L))
        oi_v[...] = jnp.where(lanes == 0, a1, a2)
        ow_v[...] = jnp.where(lanes == 0, m1, m2)
        pltpu.sync_copy(oi_v, idx_hbm)
        pltpu.sync_copy(ow_v, w_hbm)


def _mm_kernel(idx_ref, w_ref, x_ref, w0_ref, w1_ref, b0_ref, b1_ref, o_ref,
               wc_ref, bc_ref):
    del idx_ref
    t = pl.program_id(0)
    w0 = w_ref[0]
    w1 = w_ref[1]

    @pl.when(t < _JW)
    def _():
        wc_ref[:, pl.ds(t * _BW, _BW)] = (
            w0 * w0_ref[0] + w1 * w1_ref[0]).astype(jnp.bfloat16)

    @pl.when(t == 0)
    def _():
        bc_ref[...] = w0 * b0_ref[0] + w1 * b1_ref[0]

    @pl.when(t >= _JW)
    def _():
        acc = jnp.dot(x_ref[...].astype(jnp.bfloat16), wc_ref[...],
                      preferred_element_type=jnp.float32)
        o_ref[...] = acc + bc_ref[...]


@functools.partial(jax.jit, static_argnames=())
def kernel(x, gate_W, gate_b, experts_W, experts_b):
    sc_gate = pl.kernel(
        _sc_gate_kernel,
        out_type=[
            jax.ShapeDtypeStruct((_L,), jnp.int32),
            jax.ShapeDtypeStruct((_L,), jnp.float32),
        ],
        mesh=plsc.VectorSubcoreMesh(core_axis_name="c", subcore_axis_name="s",
                                    num_cores=1, num_subcores=1),
        scratch_types=[
            pltpu.VMEM((D,), jnp.float32),
            pltpu.VMEM((D * E,), jnp.float32),
            pltpu.VMEM((_L,), jnp.float32),
            pltpu.VMEM((_L,), jnp.int32),
            pltpu.VMEM((_L,), jnp.float32),
        ],
    )
    idx16, w16 = sc_gate(x, gate_W.reshape(-1),
                         jnp.pad(gate_b, (0, _L - E)))

    grid = (_JW + N // _BN,)
    out = pl.pallas_call(
        _mm_kernel,
        grid_spec=pltpu.PrefetchScalarGridSpec(
            num_scalar_prefetch=2,
            grid=grid,
            in_specs=[
                pl.BlockSpec((_BN, D),
                             lambda t, idx, w: (jnp.maximum(t - _JW, 0), 0)),
                pl.BlockSpec((1, D, _BW),
                             lambda t, idx, w: (idx[0], 0,
                                                jnp.minimum(t, _JW - 1))),
                pl.BlockSpec((1, D, _BW),
                             lambda t, idx, w: (idx[1], 0,
                                                jnp.minimum(t, _JW - 1))),
                pl.BlockSpec((1, 1, D), lambda t, idx, w: (idx[0], 0, 0)),
                pl.BlockSpec((1, 1, D), lambda t, idx, w: (idx[1], 0, 0)),
            ],
            out_specs=pl.BlockSpec((_BN, D),
                                   lambda t, idx, w: (jnp.maximum(t - _JW, 0),
                                                      0)),
            scratch_shapes=[pltpu.VMEM((D, D), jnp.bfloat16),
                            pltpu.VMEM((1, D), jnp.float32)],
        ),
        out_shape=jax.ShapeDtypeStruct((N, D), jnp.float32),
        compiler_params=pltpu.CompilerParams(
            dimension_semantics=("arbitrary",),
        ),
    )(idx16, w16, x, experts_W, experts_W,
      experts_b.reshape(E, 1, D), experts_b.reshape(E, 1, D))
    return out
